# baseline (speedup 1.0000x reference)
import functools

import jax
import jax.numpy as jnp
from jax import lax
from jax.experimental import pallas as pl
from jax.experimental.pallas import tpu as pltpu
from jax.experimental.pallas import tpu_sc as plsc

PAD_ID = 0
START_ID = 0
LANES = 16
NBUF = 3
SUB = 32


@functools.partial(jax.jit, static_argnames=("B", "S", "D"))
def _embed_lookup(ids2d, wte, wpe, B, S, D):
    NC, NS = 2, 16
    NW = NC * NS
    CH = S // NW
    nsub = B * (CH // SUB)

    mesh = plsc.VectorSubcoreMesh(core_axis_name="c", subcore_axis_name="s")

    @functools.partial(
        pl.kernel,
        mesh=mesh,
        out_type=jax.ShapeDtypeStruct((B * S, D), jnp.float32),
        scratch_types=[
            pltpu.VMEM((B, CH), jnp.int32),
            pltpu.VMEM((CH, D), jnp.float32),
        ]
        + [pltpu.VMEM((SUB, D), jnp.float32) for _ in range(NBUF)]
        + [pltpu.SemaphoreType.DMA for _ in range(2 * NBUF)],
    )
    def k(ids_hbm, wte_hbm, wpe_hbm, out_hbm, idx_v, wpe_v, *bufs_sems):
        rows = bufs_sems[:NBUF]
        gsem = bufs_sems[NBUF : 2 * NBUF]
        wsem = bufs_sems[2 * NBUF :]
        wid = lax.axis_index("s") * NC + lax.axis_index("c")
        s0 = wid * CH
        for b in range(B):
            pltpu.sync_copy(ids_hbm.at[b, pl.ds(s0, CH)], idx_v.at[b])
        pltpu.sync_copy(wpe_hbm.at[pl.ds(s0, CH), :], wpe_v)

        writes = [None] * NBUF

        def start_gather(j):
            p = j % NBUF
            if writes[p] is not None:
                writes[p].wait()
            b, h = j // (CH // SUB), j % (CH // SUB)
            return pltpu.async_copy(
                wte_hbm.at[idx_v.at[b, pl.ds(h * SUB, SUB)]], rows[p], gsem[p]
            )

        def make_add(p, h):
            def add_row(i, _):
                for jj in range(D // LANES):
                    sl = pl.ds(jj * LANES, LANES)
                    plsc.addupdate(rows[p].at[i, sl], wpe_v[h * SUB + i, sl])
                return _

            return add_row

        gathers = [None] * NBUF
        gathers[0] = start_gather(0)
        for j in range(nsub):
            p = j % NBUF
            if j + 1 < nsub:
                gathers[(j + 1) % NBUF] = start_gather(j + 1)
            gathers[p].wait()
            b, h = j // (CH // SUB), j % (CH // SUB)
            lax.fori_loop(0, SUB, make_add(p, h), 0)
            writes[p] = pltpu.async_copy(
                rows[p], out_hbm.at[pl.ds(b * S + s0 + h * SUB, SUB), :], wsem[p]
            )
        for p in range(NBUF):
            if writes[p] is not None:
                writes[p].wait()

    return k(ids2d, wte, wpe)


def kernel(encoder_hidden_states, labels, metadata, wte, wpe):
    B, S = labels.shape
    D = wte.shape[1]

    ids = jnp.concatenate(
        [jnp.full((B, 1), START_ID, labels.dtype), labels[:, :-1]], axis=1
    )
    ids = jnp.where(ids == -100, PAD_ID, ids)

    token_emb = _embed_lookup(ids, wte, wpe, B, S, D)
    token_emb = token_emb.reshape(B, S, D)

    enc_b, enc_s, _ = encoder_hidden_states.shape
    encoder_extended_attention_mask = jnp.zeros(
        (enc_b, 1, 1, enc_s), dtype=jnp.float32
    )

    return (
        encoder_hidden_states,
        token_emb,
        encoder_extended_attention_mask,
        metadata,
        ids,
        labels,
    )
